# baseline (speedup 1.0000x reference)
import functools
import jax
import jax.numpy as jnp
from jax import lax
from jax.experimental import pallas as pl
from jax.experimental.pallas import tpu as pltpu
from jax.experimental.pallas import tpu_sc as plsc

N_NODES = 100000
N_EDGES = 3200000
D = 16
BATCH = 128
EBLOCKS = N_EDGES // BATCH
BLK_PER_CHUNK = 1
CHUNK_EDGES = BLK_PER_CHUNK * BATCH
N_CHUNKS = EBLOCKS // BLK_PER_CHUNK
NC, NS = 2, 16
NW = NC * NS
NBUF = 2
ACC_STRIPE = 6248
ACC_REM = N_NODES - NS * ACC_STRIPE


@functools.partial(
    pl.kernel,
    out_type=jax.ShapeDtypeStruct((NC, N_NODES, D), jnp.float32),
    mesh=plsc.VectorSubcoreMesh(core_axis_name="c", subcore_axis_name="s"),
    compiler_params=pltpu.CompilerParams(use_tc_tiling_on_sc=False,
                                         needs_layout_passes=False),
    scratch_types=[
        pltpu.VMEM_SHARED((N_NODES, D), jnp.float32),
        pltpu.VMEM((NBUF, CHUNK_EDGES), jnp.int32),
        pltpu.VMEM((NBUF, 2, BLK_PER_CHUNK, 8, BATCH), jnp.float32),
        pltpu.VMEM((NBUF, CHUNK_EDGES, D), jnp.float32),
        pltpu.SemaphoreType.DMA((NBUF,)),
        pltpu.SemaphoreType.DMA((NBUF,)),
    ],
)
def _sc_scatter_add(edges_hbm, recv_hbm, zeros_hbm, out_hbm,
                    acc, idx_v, raw_v, stage_v, lsem, ssem):
    c = lax.axis_index("c")
    s = lax.axis_index("s")
    w = s * NC + c
    n_trips = (N_CHUNKS - w + NW - 1) // NW

    def start_loads(i, b):
        cid = w + NW * i
        e0 = cid * CHUNK_EDGES
        pltpu.async_copy(recv_hbm.at[pl.ds(e0, CHUNK_EDGES)],
                         idx_v.at[b], lsem.at[b])
        for h in range(2):
            pltpu.async_copy(
                edges_hbm.at[h, pl.ds(cid * BLK_PER_CHUNK, BLK_PER_CHUNK)],
                raw_v.at[b, h], lsem.at[b])

    def wait_loads(i, b):
        cid = w + NW * i
        e0 = cid * CHUNK_EDGES
        pltpu.make_async_copy(recv_hbm.at[pl.ds(e0, CHUNK_EDGES)],
                              idx_v.at[b], lsem.at[b]).wait()
        for h in range(2):
            pltpu.make_async_copy(
                edges_hbm.at[h, pl.ds(cid * BLK_PER_CHUNK, BLK_PER_CHUNK)],
                raw_v.at[b, h], lsem.at[b]).wait()

    iota16 = lax.iota(jnp.int32, 16)

    def transpose_chunk(b):
        for blk in range(BLK_PER_CHUNK):
            blk_v = jnp.full((16,), blk, jnp.int32)

            def dbody(d, carry, blk=blk, blk_v=blk_v):
                fcol = (iota16 + d) & 15
                fh = fcol >> 3
                f8 = fcol & 7
                for grp in range(BATCH // 16):
                    rows = iota16 + grp * 16
                    x = plsc.load_gather(raw_v.at[b], [fh, blk_v, f8, rows])
                    plsc.store_scatter(
                        stage_v.at[b], [rows + blk * BATCH, fcol], x)
                return carry

            lax.fori_loop(0, D, dbody, 0)

    def fire_scatters(b):
        for j in range(BLK_PER_CHUNK):
            pltpu.async_copy(stage_v.at[b, pl.ds(j * BATCH, BATCH)],
                             acc.at[idx_v.at[b, pl.ds(j * BATCH, BATCH)]],
                             ssem.at[b], add=True)

    def drain_scatters(b):
        for j in range(BLK_PER_CHUNK):
            pltpu.make_async_copy(stage_v.at[b, pl.ds(j * BATCH, BATCH)],
                                  acc.at[idx_v.at[b, pl.ds(j * BATCH, BATCH)]],
                                  ssem.at[b]).wait()

    start_loads(0, 0)
    pltpu.sync_copy(
        zeros_hbm.at[pl.ds(0, ACC_STRIPE)],
        acc.at[pl.ds(s * ACC_STRIPE, ACC_STRIPE)],
    )

    @pl.when(s == NS - 1)
    def _zero_tail():
        pltpu.sync_copy(
            zeros_hbm.at[pl.ds(0, ACC_REM)],
            acc.at[pl.ds(NS * ACC_STRIPE, ACC_REM)],
        )

    plsc.subcore_barrier()

    def outer(t, carry):
        for b in range(NBUF):
            i = t * NBUF + b

            @pl.when((i >= NBUF - 1) & (i - (NBUF - 1) < n_trips))
            def _drain():
                drain_scatters((b + 1) % NBUF)

            @pl.when(i + 1 < n_trips)
            def _prefetch():
                start_loads(i + 1, (b + 1) % NBUF)

            @pl.when(i < n_trips)
            def _process():
                wait_loads(i, b)
                transpose_chunk(b)
                fire_scatters(b)

        return carry

    outer_trips = ((N_CHUNKS + NW - 1) // NW + 2 * (NBUF - 1)) // NBUF + 1
    lax.fori_loop(0, outer_trips, outer, 0)
    plsc.subcore_barrier()

    pltpu.sync_copy(
        acc.at[pl.ds(s * ACC_STRIPE, ACC_STRIPE)],
        out_hbm.at[c, pl.ds(s * ACC_STRIPE, ACC_STRIPE)],
    )

    @pl.when(s == NS - 1)
    def _write_tail():
        pltpu.sync_copy(
            acc.at[pl.ds(NS * ACC_STRIPE, ACC_REM)],
            out_hbm.at[c, pl.ds(NS * ACC_STRIPE, ACC_REM)],
        )


def _combine_body(p_ref, o_ref):
    o_ref[...] = p_ref[0] + p_ref[1]


def kernel(edges, nodes, receivers, senders):
    edges_phys = edges.reshape(EBLOCKS, BATCH, 2, 8).transpose(2, 0, 3, 1)
    zeros = jnp.zeros((ACC_STRIPE, D), jnp.float32)
    partials = _sc_scatter_add(edges_phys, receivers, zeros)

    flat = partials.reshape(NC, (N_NODES * D) // 128, 128)
    n_rows = flat.shape[1]
    out = pl.pallas_call(
        _combine_body,
        out_shape=jax.ShapeDtypeStruct((n_rows, 128), jnp.float32),
    )(flat)
    return out.reshape(N_NODES, D)


# speedup vs baseline: 1.2405x; 1.2405x over previous
import functools
import jax
import jax.numpy as jnp
from jax import lax
from jax.experimental import pallas as pl
from jax.experimental.pallas import tpu as pltpu
from jax.experimental.pallas import tpu_sc as plsc

N_NODES = 100000
N_EDGES = 3200000
D = 16
BATCH = 128
EBLOCKS = N_EDGES // BATCH
BLK_PER_CHUNK = 2
CHUNK_EDGES = BLK_PER_CHUNK * BATCH
N_CHUNKS = EBLOCKS // BLK_PER_CHUNK
NC, NS = 2, 16
NW = NC * NS
NBUF = 3
ACC_STRIPE = 6248
ACC_REM = N_NODES - NS * ACC_STRIPE


@functools.partial(
    pl.kernel,
    out_type=jax.ShapeDtypeStruct((NC, N_NODES, D), jnp.float32),
    mesh=plsc.VectorSubcoreMesh(core_axis_name="c", subcore_axis_name="s"),
    compiler_params=pltpu.CompilerParams(use_tc_tiling_on_sc=False,
                                         needs_layout_passes=False),
    scratch_types=[
        pltpu.VMEM_SHARED((N_NODES, D), jnp.float32),
        pltpu.VMEM((NBUF, CHUNK_EDGES), jnp.int32),
        pltpu.VMEM((NBUF, 2, BLK_PER_CHUNK, 8, BATCH), jnp.float32),
        pltpu.VMEM((NBUF, CHUNK_EDGES, D), jnp.float32),
        pltpu.SemaphoreType.DMA((NBUF,)),
        pltpu.SemaphoreType.DMA((NBUF,)),
    ],
)
def _sc_scatter_add(edges_hbm, recv_hbm, zeros_hbm, out_hbm,
                    acc, idx_v, raw_v, stage_v, lsem, ssem):
    c = lax.axis_index("c")
    s = lax.axis_index("s")
    w = s * NC + c
    n_trips = (N_CHUNKS - w + NW - 1) // NW

    def start_loads(i, b):
        cid = w + NW * i
        e0 = cid * CHUNK_EDGES
        pltpu.async_copy(recv_hbm.at[pl.ds(e0, CHUNK_EDGES)],
                         idx_v.at[b], lsem.at[b])
        for h in range(2):
            pltpu.async_copy(
                edges_hbm.at[h, pl.ds(cid * BLK_PER_CHUNK, BLK_PER_CHUNK)],
                raw_v.at[b, h], lsem.at[b])

    def wait_loads(i, b):
        cid = w + NW * i
        e0 = cid * CHUNK_EDGES
        pltpu.make_async_copy(recv_hbm.at[pl.ds(e0, CHUNK_EDGES)],
                              idx_v.at[b], lsem.at[b]).wait()
        for h in range(2):
            pltpu.make_async_copy(
                edges_hbm.at[h, pl.ds(cid * BLK_PER_CHUNK, BLK_PER_CHUNK)],
                raw_v.at[b, h], lsem.at[b]).wait()

    iota16 = lax.iota(jnp.int32, 16)

    def transpose_chunk(b):
        for blk in range(BLK_PER_CHUNK):
            for grp in range(BATCH // 16):
                e0 = grp * 16
                rows = blk * BATCH + e0 + iota16
                for h in range(2):
                    for f in range(8):
                        x = raw_v[b, h, blk, f, pl.ds(e0, 16)]
                        plsc.store_scatter(
                            stage_v.at[b],
                            [rows, jnp.full((16,), 8 * h + f, jnp.int32)],
                            x)

    def fire_scatters(b):
        for j in range(BLK_PER_CHUNK):
            pltpu.async_copy(stage_v.at[b, pl.ds(j * BATCH, BATCH)],
                             acc.at[idx_v.at[b, pl.ds(j * BATCH, BATCH)]],
                             ssem.at[b], add=True)

    def drain_scatters(b):
        for j in range(BLK_PER_CHUNK):
            pltpu.make_async_copy(stage_v.at[b, pl.ds(j * BATCH, BATCH)],
                                  acc.at[idx_v.at[b, pl.ds(j * BATCH, BATCH)]],
                                  ssem.at[b]).wait()

    start_loads(0, 0)
    pltpu.sync_copy(
        zeros_hbm.at[pl.ds(0, ACC_STRIPE)],
        acc.at[pl.ds(s * ACC_STRIPE, ACC_STRIPE)],
    )

    @pl.when(s == NS - 1)
    def _zero_tail():
        pltpu.sync_copy(
            zeros_hbm.at[pl.ds(0, ACC_REM)],
            acc.at[pl.ds(NS * ACC_STRIPE, ACC_REM)],
        )

    plsc.subcore_barrier()

    def outer(t, carry):
        for b in range(NBUF):
            i = t * NBUF + b

            @pl.when((i >= NBUF - 1) & (i - (NBUF - 1) < n_trips))
            def _drain():
                drain_scatters((b + 1) % NBUF)

            @pl.when(i + 1 < n_trips)
            def _prefetch():
                start_loads(i + 1, (b + 1) % NBUF)

            @pl.when(i < n_trips)
            def _process():
                wait_loads(i, b)
                transpose_chunk(b)
                fire_scatters(b)

        return carry

    outer_trips = ((N_CHUNKS + NW - 1) // NW + 2 * (NBUF - 1)) // NBUF + 1
    lax.fori_loop(0, outer_trips, outer, 0)
    plsc.subcore_barrier()

    pltpu.sync_copy(
        acc.at[pl.ds(s * ACC_STRIPE, ACC_STRIPE)],
        out_hbm.at[c, pl.ds(s * ACC_STRIPE, ACC_STRIPE)],
    )

    @pl.when(s == NS - 1)
    def _write_tail():
        pltpu.sync_copy(
            acc.at[pl.ds(NS * ACC_STRIPE, ACC_REM)],
            out_hbm.at[c, pl.ds(NS * ACC_STRIPE, ACC_REM)],
        )


def _combine_body(p_ref, o_ref):
    o_ref[...] = p_ref[0] + p_ref[1]


def kernel(edges, nodes, receivers, senders):
    edges_phys = edges.reshape(EBLOCKS, BATCH, 2, 8).transpose(2, 0, 3, 1)
    zeros = jnp.zeros((ACC_STRIPE, D), jnp.float32)
    partials = _sc_scatter_add(edges_phys, receivers, zeros)

    flat = partials.reshape(NC, (N_NODES * D) // 128, 128)
    n_rows = flat.shape[1]
    out = pl.pallas_call(
        _combine_body,
        out_shape=jax.ShapeDtypeStruct((n_rows, 128), jnp.float32),
    )(flat)
    return out.reshape(N_NODES, D)
